# baseline (speedup 1.0000x reference)
import functools

import jax
import jax.numpy as jnp
import numpy as np
from jax import lax
from jax.experimental import pallas as pl
from jax.experimental.pallas import tpu as pltpu
from jax.experimental.pallas import tpu_sc as plsc

NBINS = 256
NB2 = NBINS * NBINS
H = 512
W = 512
NIMG = 12
NPR = H - 1
MAXSLOTS = 4
CH = 64
BUFROWS = CH + 1

_PI = np.float32(np.pi)
_PI_2 = np.float32(np.pi / 2.0)
_C1 = np.float32(1.0)
_C3 = np.float32(-1.0 / 6.0)
_C5 = np.float32(1.0 / 120.0)
_C7 = np.float32(-1.0 / 5040.0)
_C9 = np.float32(1.0 / 362880.0)


def _bin16(v):
    ts = v * np.float32(NBINS) + np.float32(NBINS - 0.5)
    ti = ts.astype(jnp.int32)
    f = ts - ti.astype(jnp.float32)
    i0 = ti - NBINS
    x = f * _PI - _PI_2
    x2 = x * x
    s = x * (_C1 + x2 * (_C3 + x2 * (_C5 + x2 * (_C7 + x2 * _C9))))
    w0 = np.float32(0.5) - np.float32(0.5) * s
    return i0, w0


def _zero_hist(hist):
    @pl.loop(0, NB2, step=16, unroll=8)
    def _zero(i):
        hist[pl.ds(i, 16)] = jnp.zeros((16,), jnp.float32)


def _segment(x_ref, out_ref, hist, buf, im, start, end, slot):
    img_base = im * (H * W)
    nr = end - start
    nchunks = (nr + CH - 1) // CH

    @pl.loop(0, nchunks)
    def _chunk(k):
        ps = start + k * CH
        npair = jnp.minimum(ps + CH, end) - ps
        ls = jnp.minimum(ps, H - BUFROWS)
        off = ps - ls
        pltpu.sync_copy(x_ref.at[pl.ds(img_base + ls * W, BUFROWS * W)], buf)

        @plsc.parallel_loop(0, W // 16, unroll=2)
        def _colgroup(cg):
            cb = cg * 16
            v0 = buf[pl.ds(off * W + cb, 16)]
            i0a, w0a = _bin16(v0)
            init = (
                jnp.maximum(i0a, 0) << 8,
                jnp.minimum(i0a + 1, NBINS - 1) << 8,
                w0a,
                np.float32(1.0) - w0a,
            )

            @plsc.parallel_loop(0, CH, unroll=4, carry=init)
            def _pair(i, carry):
                ia0s, ia1s, wa0, wa1 = carry
                valid = jnp.broadcast_to(i < npair, (16,))
                v = buf[pl.ds((off + 1 + i) * W + cb, 16)]
                i0, wb0 = _bin16(v)
                wb1 = np.float32(1.0) - wb0
                ib0 = jnp.maximum(i0, 0)
                ib1 = jnp.minimum(i0 + 1, NBINS - 1)
                plsc.addupdate_scatter(hist, [ia0s + ib0], wa0 * wb0, mask=valid)
                plsc.addupdate_scatter(hist, [ia0s + ib1], wa0 * wb1, mask=valid)
                plsc.addupdate_scatter(hist, [ia1s + ib0], wa1 * wb0, mask=valid)
                plsc.addupdate_scatter(hist, [ia1s + ib1], wa1 * wb1, mask=valid)
                return (ib0 << 8, ib1 << 8, wb0, wb1)

    pltpu.sync_copy(hist, out_ref.at[pl.ds((im * MAXSLOTS + slot) * NB2, NB2)])


def _sc_body(x_ref, out_ref, hist, buf):
    wid = lax.axis_index("s") * 2 + lax.axis_index("c")
    g = wid // 8
    j = wid % 8
    g0 = (j * 1533) // 8
    g1 = ((j + 1) * 1533) // 8
    q0 = g0 // NPR
    r0 = g0 - q0 * NPR
    e0 = jnp.minimum(NPR, g1 - q0 * NPR)
    qe = (g1 - 1) // NPR
    has2 = qe > q0

    _zero_hist(hist)

    @pl.when(jnp.logical_or(j == 0, j == 7))
    def _zero_duty():
        zim = 3 * g + jnp.where(j == 0, 0, 2)
        pltpu.sync_copy(hist, out_ref.at[pl.ds((zim * MAXSLOTS + 3) * NB2, NB2)])

    slot0 = j - (4088 * q0) // 1533
    _segment(x_ref, out_ref, hist, buf, 3 * g + q0, r0, e0, slot0)

    @pl.when(has2)
    def _second():
        _zero_hist(hist)
        slot1 = j - (4088 * qe) // 1533
        _segment(x_ref, out_ref, hist, buf, 3 * g + qe, 0, g1 - qe * NPR, slot1)


_sc_hist = functools.partial(
    pl.kernel,
    out_type=jax.ShapeDtypeStruct((NIMG * MAXSLOTS * NB2,), jnp.float32),
    mesh=plsc.VectorSubcoreMesh(core_axis_name="c", subcore_axis_name="s"),
    compiler_params=pltpu.CompilerParams(needs_layout_passes=False),
    scratch_types=[
        pltpu.VMEM((NB2,), jnp.float32),
        pltpu.VMEM((BUFROWS * W,), jnp.float32),
    ],
)(_sc_body)


def _merge_body(p_ref, o_ref):
    s = p_ref[0, 0:1, :] + p_ref[0, 1:2, :] + p_ref[0, 2:3, :] + p_ref[0, 3:4, :]
    o_ref[0] = s / jnp.max(s)


def kernel(X):
    x_flat = X.reshape(-1)
    partials = _sc_hist(x_flat).reshape(NIMG, MAXSLOTS, NB2)
    out = pl.pallas_call(
        _merge_body,
        grid=(NIMG,),
        in_specs=[pl.BlockSpec((1, MAXSLOTS, NB2), lambda i: (i, 0, 0))],
        out_specs=pl.BlockSpec((1, 1, NB2), lambda i: (i, 0, 0)),
        out_shape=jax.ShapeDtypeStruct((NIMG, 1, NB2), jnp.float32),
    )(partials)
    return out.reshape(4, 3, NBINS, NBINS)


# speedup vs baseline: 1.0830x; 1.0830x over previous
import functools

import jax
import jax.numpy as jnp
import numpy as np
from jax import lax
from jax.experimental import pallas as pl
from jax.experimental.pallas import tpu as pltpu
from jax.experimental.pallas import tpu_sc as plsc

NBINS = 256
NB2 = NBINS * NBINS
H = 512
W = 512
NIMG = 12
NPR = H - 1
MAXSLOTS = 4
CH = 64
BUFROWS = CH + 1

_PI = np.float32(np.pi)
_PI_2 = np.float32(np.pi / 2.0)
_A = np.float32(0.4999447113271096)
_B = np.float32(-0.08295371865075976)
_C = np.float32(0.003785384589305135)


def _bin16(v):
    ts = v * np.float32(NBINS) + np.float32(NBINS - 0.5)
    ti = ts.astype(jnp.int32)
    f = ts - ti.astype(jnp.float32)
    i0 = ti - NBINS
    x = f * _PI - _PI_2
    u = x * x
    h = x * (_A + u * (_B + _C * u))
    w0 = np.float32(0.5) - h
    return i0, w0


def _zero_hist(hist):
    @pl.loop(0, NB2, step=16, unroll=8)
    def _zero(i):
        hist[pl.ds(i, 16)] = jnp.zeros((16,), jnp.float32)


def _segment(x_ref, out_ref, hist, buf, im, start, end, slot):
    img_base = im * (H * W)
    nr = end - start
    nchunks = (nr + CH - 1) // CH

    @pl.loop(0, nchunks)
    def _chunk(k):
        ps = start + k * CH
        npair = jnp.minimum(ps + CH, end) - ps
        ls = jnp.minimum(ps, H - BUFROWS)
        off = ps - ls
        pltpu.sync_copy(x_ref.at[pl.ds(img_base + ls * W, BUFROWS * W)], buf)

        @pl.loop(0, W // 16)
        def _colgroup(cg):
            cb = cg * 16
            v0 = buf[pl.ds(off * W + cb, 16)]
            i0a, w0a = _bin16(v0)
            init = (
                jnp.maximum(i0a, 0) << 8,
                jnp.minimum(i0a + 1, NBINS - 1) << 8,
                w0a,
                np.float32(1.0) - w0a,
            )

            @plsc.parallel_loop(0, CH, unroll=4, carry=init)
            def _pair(i, carry):
                ia0s, ia1s, wa0, wa1 = carry
                valid = jnp.broadcast_to(i < npair, (16,))
                v = buf[pl.ds((off + 1 + i) * W + cb, 16)]
                i0, wb0 = _bin16(v)
                wb1 = np.float32(1.0) - wb0
                ib0 = jnp.maximum(i0, 0)
                ib1 = jnp.minimum(i0 + 1, NBINS - 1)
                plsc.addupdate_scatter(hist, [ia0s + ib0], wa0 * wb0, mask=valid)
                plsc.addupdate_scatter(hist, [ia0s + ib1], wa0 * wb1, mask=valid)
                plsc.addupdate_scatter(hist, [ia1s + ib0], wa1 * wb0, mask=valid)
                plsc.addupdate_scatter(hist, [ia1s + ib1], wa1 * wb1, mask=valid)
                return (ib0 << 8, ib1 << 8, wb0, wb1)

    pltpu.sync_copy(hist, out_ref.at[pl.ds((im * MAXSLOTS + slot) * NB2, NB2)])


def _sc_body(x_ref, out_ref, hist, buf):
    wid = lax.axis_index("s") * 2 + lax.axis_index("c")
    g = wid // 8
    j = wid % 8
    g0 = (j * 1533) // 8
    g1 = ((j + 1) * 1533) // 8
    q0 = g0 // NPR
    r0 = g0 - q0 * NPR
    e0 = jnp.minimum(NPR, g1 - q0 * NPR)
    qe = (g1 - 1) // NPR
    has2 = qe > q0

    _zero_hist(hist)

    @pl.when(jnp.logical_or(j == 0, j == 7))
    def _zero_duty():
        zim = 3 * g + jnp.where(j == 0, 0, 2)
        pltpu.sync_copy(hist, out_ref.at[pl.ds((zim * MAXSLOTS + 3) * NB2, NB2)])

    slot0 = j - (4088 * q0) // 1533
    _segment(x_ref, out_ref, hist, buf, 3 * g + q0, r0, e0, slot0)

    @pl.when(has2)
    def _second():
        _zero_hist(hist)
        slot1 = j - (4088 * qe) // 1533
        _segment(x_ref, out_ref, hist, buf, 3 * g + qe, 0, g1 - qe * NPR, slot1)


_sc_hist = functools.partial(
    pl.kernel,
    out_type=jax.ShapeDtypeStruct((NIMG * MAXSLOTS * NB2,), jnp.float32),
    mesh=plsc.VectorSubcoreMesh(core_axis_name="c", subcore_axis_name="s"),
    compiler_params=pltpu.CompilerParams(needs_layout_passes=False),
    scratch_types=[
        pltpu.VMEM((NB2,), jnp.float32),
        pltpu.VMEM((BUFROWS * W,), jnp.float32),
    ],
)(_sc_body)


def _merge_body(p_ref, o_ref):
    s = p_ref[0, 0:1, :] + p_ref[0, 1:2, :] + p_ref[0, 2:3, :] + p_ref[0, 3:4, :]
    o_ref[0] = s / jnp.max(s)


def kernel(X):
    x_flat = X.reshape(-1)
    partials = _sc_hist(x_flat).reshape(NIMG, MAXSLOTS, NB2)
    out = pl.pallas_call(
        _merge_body,
        grid=(NIMG,),
        in_specs=[pl.BlockSpec((1, MAXSLOTS, NB2), lambda i: (i, 0, 0))],
        out_specs=pl.BlockSpec((1, 1, NB2), lambda i: (i, 0, 0)),
        out_shape=jax.ShapeDtypeStruct((NIMG, 1, NB2), jnp.float32),
    )(partials)
    return out.reshape(4, 3, NBINS, NBINS)
